# baseline (speedup 1.0000x reference)
import functools

import jax
import jax.numpy as jnp
from jax import lax
from jax.experimental import pallas as pl
from jax.experimental.pallas import tpu as pltpu
from jax.experimental.pallas import tpu_sc as plsc

VOCAB = 1000000
D = 64
B_FLAT = 4096 * 200

NW = 32
CHUNK = 128
K = 4
GROUP = CHUNK * K
N_GROUPS = B_FLAT // (NW * GROUP)


def _emb_kernel(idx_hbm, table_hbm, out_hbm, idx_v, rows_v, gsem, osem):
    wid = lax.axis_index("s") * 2 + lax.axis_index("c")

    pltpu.sync_copy(idx_hbm.at[wid], idx_v)

    def fire_gathers(g, slot):
        for b in range(K):
            pltpu.async_copy(
                table_hbm.at[idx_v.at[g, b]],
                rows_v.at[slot, pl.ds(b * CHUNK, CHUNK)],
                gsem,
            )

    def drain_gathers(slot):
        for b in range(K):
            pltpu.make_async_copy(
                out_hbm.at[wid, 0, pl.ds(b * CHUNK, CHUNK)],
                rows_v.at[slot, pl.ds(b * CHUNK, CHUNK)],
                gsem,
            ).wait()

    def fire_write(g, slot):
        pltpu.async_copy(rows_v.at[slot], out_hbm.at[wid, g], osem)

    def drain_write():
        pltpu.make_async_copy(rows_v.at[0], out_hbm.at[wid, 0], osem).wait()

    fire_gathers(0, 0)

    def body(g, carry):
        slot = lax.rem(g, 2)
        nslot = lax.rem(g + 1, 2)

        @pl.when(g >= 1)
        def _():
            drain_write()

        @pl.when(g + 1 < N_GROUPS)
        def _():
            fire_gathers(g + 1, nslot)

        drain_gathers(slot)
        fire_write(g, slot)
        return carry

    lax.fori_loop(0, N_GROUPS, body, 0)
    drain_write()


@functools.partial(
    pl.kernel,
    out_type=jax.ShapeDtypeStruct((NW, N_GROUPS, GROUP, D), jnp.float32),
    scratch_types=[
        pltpu.VMEM((N_GROUPS, K, CHUNK), jnp.int32),
        pltpu.VMEM((2, GROUP, D), jnp.float32),
        pltpu.SemaphoreType.DMA,
        pltpu.SemaphoreType.DMA,
    ],
    mesh=plsc.VectorSubcoreMesh(core_axis_name="c", subcore_axis_name="s"),
    compiler_params=pltpu.CompilerParams(use_tc_tiling_on_sc=False),
)
def _emb(idx_hbm, table_hbm, out_hbm, idx_v, rows_v, gsem, osem):
    _emb_kernel(idx_hbm, table_hbm, out_hbm, idx_v, rows_v, gsem, osem)


def kernel(x, weights):
    b, l = x.shape
    idx = x.astype(jnp.int32).reshape(NW, N_GROUPS, K, CHUNK)
    out = _emb(idx, weights)
    return out.reshape(b, l, D)


# speedup vs baseline: 1.0051x; 1.0051x over previous
import functools

import jax
import jax.numpy as jnp
from jax import lax
from jax.experimental import pallas as pl
from jax.experimental.pallas import tpu as pltpu
from jax.experimental.pallas import tpu_sc as plsc

VOCAB = 1000000
D = 64
B_FLAT = 4096 * 200

NW = 32
GROUP = 512
NSLOT = 3
N_GROUPS = B_FLAT // (NW * GROUP)


def _emb_kernel(idx_hbm, table_hbm, out_hbm, idx_v, rows_v, gsem, osem):
    wid = lax.axis_index("s") * 2 + lax.axis_index("c")

    pltpu.sync_copy(idx_hbm.at[wid], idx_v)

    def fire_gather(g, slot):
        pltpu.async_copy(table_hbm.at[idx_v.at[g]], rows_v.at[slot], gsem)

    def drain_gather():
        pltpu.make_async_copy(out_hbm.at[wid, 0], rows_v.at[0], gsem).wait()

    def fire_write(g, slot):
        pltpu.async_copy(rows_v.at[slot], out_hbm.at[wid, g], osem)

    def drain_write():
        pltpu.make_async_copy(rows_v.at[0], out_hbm.at[wid, 0], osem).wait()

    fire_gather(0, 0)
    fire_gather(1, 1)

    def body(g, carry):
        slot = lax.rem(g, NSLOT)
        nslot = lax.rem(g + 2, NSLOT)

        @pl.when(g >= 1)
        def _():
            drain_write()

        @pl.when(g + 2 < N_GROUPS)
        def _():
            fire_gather(g + 2, nslot)

        drain_gather()
        fire_write(g, slot)
        return carry

    lax.fori_loop(0, N_GROUPS, body, 0)
    drain_write()


@functools.partial(
    pl.kernel,
    out_type=jax.ShapeDtypeStruct((NW, N_GROUPS, GROUP, D), jnp.float32),
    scratch_types=[
        pltpu.VMEM((N_GROUPS, GROUP), jnp.int32),
        pltpu.VMEM((NSLOT, GROUP, D), jnp.float32),
        pltpu.SemaphoreType.DMA,
        pltpu.SemaphoreType.DMA,
    ],
    mesh=plsc.VectorSubcoreMesh(core_axis_name="c", subcore_axis_name="s"),
    compiler_params=pltpu.CompilerParams(use_tc_tiling_on_sc=False),
)
def _emb(idx_hbm, table_hbm, out_hbm, idx_v, rows_v, gsem, osem):
    _emb_kernel(idx_hbm, table_hbm, out_hbm, idx_v, rows_v, gsem, osem)


def kernel(x, weights):
    b, l = x.shape
    idx = x.astype(jnp.int32).reshape(NW, N_GROUPS, GROUP)
    out = _emb(idx, weights)
    return out.reshape(b, l, D)


# speedup vs baseline: 1.0051x; 1.0000x over previous
import functools

import jax
import jax.numpy as jnp
from jax import lax
from jax.experimental import pallas as pl
from jax.experimental.pallas import tpu as pltpu
from jax.experimental.pallas import tpu_sc as plsc

VOCAB = 1000000
D = 64
BATCH = 4096
HIST = 200

NW = 32
ROWS_PW = BATCH // NW
NSLOT = 3


def _emb_kernel(idx_hbm, table_hbm, out_hbm, idx_v, rows_v, gsem, osem):
    wid = lax.axis_index("s") * 2 + lax.axis_index("c")
    base = wid * ROWS_PW

    pltpu.sync_copy(idx_hbm.at[pl.ds(base, ROWS_PW)], idx_v)

    def fire_gather(i, slot):
        pltpu.async_copy(table_hbm.at[idx_v.at[i]], rows_v.at[slot], gsem)

    def drain_gather():
        pltpu.make_async_copy(out_hbm.at[0], rows_v.at[0], gsem).wait()

    def fire_write(i, slot):
        pltpu.async_copy(rows_v.at[slot], out_hbm.at[base + i], osem)

    def drain_write():
        pltpu.make_async_copy(rows_v.at[0], out_hbm.at[0], osem).wait()

    fire_gather(0, 0)
    fire_gather(1, 1)

    def body(i, carry):
        slot = lax.rem(i, NSLOT)
        nslot = lax.rem(i + 2, NSLOT)

        @pl.when(i >= 1)
        def _():
            drain_write()

        @pl.when(i + 2 < ROWS_PW)
        def _():
            fire_gather(i + 2, nslot)

        drain_gather()
        fire_write(i, slot)
        return carry

    lax.fori_loop(0, ROWS_PW, body, 0)
    drain_write()


@functools.partial(
    pl.kernel,
    out_type=jax.ShapeDtypeStruct((BATCH, HIST, D), jnp.float32),
    scratch_types=[
        pltpu.VMEM((ROWS_PW, HIST), jnp.int32),
        pltpu.VMEM((NSLOT, HIST, D), jnp.float32),
        pltpu.SemaphoreType.DMA,
        pltpu.SemaphoreType.DMA,
    ],
    mesh=plsc.VectorSubcoreMesh(core_axis_name="c", subcore_axis_name="s"),
    compiler_params=pltpu.CompilerParams(use_tc_tiling_on_sc=False),
)
def _emb(idx_hbm, table_hbm, out_hbm, idx_v, rows_v, gsem, osem):
    _emb_kernel(idx_hbm, table_hbm, out_hbm, idx_v, rows_v, gsem, osem)


def kernel(x, weights):
    return _emb(x.astype(jnp.int32), weights)


# speedup vs baseline: 1.2263x; 1.2201x over previous
import functools

import jax
import jax.numpy as jnp
from jax import lax
from jax.experimental import pallas as pl
from jax.experimental.pallas import tpu as pltpu
from jax.experimental.pallas import tpu_sc as plsc

VOCAB = 1000000
D = 64
DPAD = 128
BATCH = 4096
HIST = 200
B_FLAT = BATCH * HIST

NW = 32
CHUNK = 256
NSLOT = 3
N_PW = B_FLAT // NW
N_CHUNKS = N_PW // CHUNK


def _emb_kernel(idx_hbm, table_hbm, out_hbm, idx_v, rows_v, gsem, osem):
    wid = lax.axis_index("s") * 2 + lax.axis_index("c")
    base = wid * N_PW

    pltpu.sync_copy(idx_hbm.at[pl.ds(base, N_PW)], idx_v)

    def fire_gather(i, slot):
        pltpu.async_copy(
            table_hbm.at[idx_v.at[pl.ds(i * CHUNK, CHUNK)]],
            rows_v.at[slot],
            gsem,
        )

    def drain_gather():
        pltpu.make_async_copy(
            out_hbm.at[pl.ds(0, CHUNK)], rows_v.at[0], gsem
        ).wait()

    def fire_write(i, slot):
        pltpu.async_copy(
            rows_v.at[slot], out_hbm.at[pl.ds(base + i * CHUNK, CHUNK)], osem
        )

    def drain_write():
        pltpu.make_async_copy(
            rows_v.at[0], out_hbm.at[pl.ds(0, CHUNK)], osem
        ).wait()

    fire_gather(0, 0)
    fire_gather(1, 1)

    def body(i, carry):
        slot = lax.rem(i, NSLOT)
        nslot = lax.rem(i + 2, NSLOT)

        @pl.when(i >= 1)
        def _():
            drain_write()

        @pl.when(i + 2 < N_CHUNKS)
        def _():
            fire_gather(i + 2, nslot)

        drain_gather()
        fire_write(i, slot)
        return carry

    lax.fori_loop(0, N_CHUNKS, body, 0)
    drain_write()


@functools.partial(
    pl.kernel,
    out_type=jax.ShapeDtypeStruct((B_FLAT, DPAD), jnp.float32),
    scratch_types=[
        pltpu.VMEM((N_PW,), jnp.int32),
        pltpu.VMEM((NSLOT, CHUNK, DPAD), jnp.float32),
        pltpu.SemaphoreType.DMA,
        pltpu.SemaphoreType.DMA,
    ],
    mesh=plsc.VectorSubcoreMesh(core_axis_name="c", subcore_axis_name="s"),
    compiler_params=pltpu.CompilerParams(use_tc_tiling_on_sc=True),
)
def _emb(idx_hbm, table_hbm, out_hbm, idx_v, rows_v, gsem, osem):
    _emb_kernel(idx_hbm, table_hbm, out_hbm, idx_v, rows_v, gsem, osem)


def kernel(x, weights):
    wpad = jnp.pad(weights, ((0, 0), (0, DPAD - D)))
    xf = x.astype(jnp.int32).reshape(B_FLAT)
    outp = _emb(xf, wpad)
    return outp.reshape(BATCH, HIST, DPAD)[..., :D]
